# baseline (speedup 1.0000x reference)
import functools

import jax
import jax.numpy as jnp
from jax.experimental import pallas as pl
from jax.experimental.pallas import tpu as pltpu


_BF16 = jnp.bfloat16


def _cheb1_body(a_ref, x0t_ref, out_ref):
    out_ref[0] = jnp.dot(a_ref[0].astype(_BF16), x0t_ref[...].astype(_BF16),
                         preferred_element_type=jnp.float32)


def _cheb2_body(a_ref, x1t_ref, x0t_ref, p_ref, *, bm, nsup):
    r = pl.program_id(0)
    i = pl.program_id(1)
    x1t = x1t_ref[0]
    rows = x1t_ref[0, pl.ds(r * bm, bm), :]
    acc = rows + 2.0 * jnp.dot(a_ref[0].astype(_BF16), x1t.astype(_BF16),
                               preferred_element_type=jnp.float32)

    @pl.when(i == 0)
    def _():
        p_ref[...] = acc + (1.0 - nsup) * x0t_ref[...]

    @pl.when(i != 0)
    def _():
        p_ref[...] += acc


def _stats_body(z_ref, c_ref, *, bi, bj, n):
    j = pl.program_id(0)
    zj = z_ref[pl.ds(j * bj, bj), :]

    zjh = zj.astype(_BF16)

    def body(k, carry):
        m, dsum = carry
        zi = z_ref[pl.ds(k * bi, bi), :]
        tile = jax.lax.dot_general(
            zi.astype(_BF16), zjh, (((1,), (1,)), ((), ())),
            preferred_element_type=jnp.float32)
        tile = jnp.maximum(tile, 0.0)
        tm = jnp.max(tile, axis=0, keepdims=True)
        m_new = jnp.maximum(m, tm)
        dsum = dsum * jnp.exp(m - m_new) + jnp.sum(
            jnp.exp(tile - m_new), axis=0, keepdims=True)
        return m_new, dsum

    m0 = jnp.zeros((1, bj), jnp.float32)
    d0 = jnp.zeros((1, bj), jnp.float32)
    m, dsum = jax.lax.fori_loop(0, n // bi, body, (m0, d0))
    c_ref[...] = m + jnp.log(dsum)


def _final_body(z_ref, c_ref, x0t_ref, p_ref, w_ref, out_ref,
                *, bi, bj, n, batch, d):
    r = pl.program_id(0)
    zi = z_ref[pl.ds(r * bi, bi), :]
    bd = batch * d

    zih = zi.astype(_BF16)

    def body(k, acc):
        zj = z_ref[pl.ds(k * bj, bj), :]
        tile = jax.lax.dot_general(
            zih, zj.astype(_BF16), (((1,), (1,)), ((), ())),
            preferred_element_type=jnp.float32)
        tile = jnp.maximum(tile, 0.0)
        e = jnp.exp(tile - c_ref[:, pl.ds(k * bj, bj)])
        v = x0t_ref[pl.ds(k * bj, bj), :]
        return acc + jnp.dot(e.astype(_BF16), v.astype(_BF16),
                             preferred_element_type=jnp.float32)

    xz = jax.lax.fori_loop(0, n // bj, body,
                           jnp.zeros((bi, bd), jnp.float32))
    s = xz + p_ref[...]
    w = w_ref[...]
    for b in range(batch):
        out_ref[b] = jnp.dot(s[:, b * d:(b + 1) * d], w,
                             preferred_element_type=jnp.float32)


def kernel(A, X, Z, W):
    nsup, n, _ = A.shape
    batch, d, _ = X.shape
    bd = batch * d
    out_f = W.shape[1]

    X0T = X.reshape(bd, n).T

    BM = 512
    BI = 512
    BJ = 512
    nb = n // BM

    x1t = pl.pallas_call(
        _cheb1_body,
        grid=(nsup, nb),
        in_specs=[
            pl.BlockSpec((1, BM, n), lambda i, r: (i, r, 0)),
            pl.BlockSpec((n, bd), lambda i, r: (0, 0)),
        ],
        out_specs=pl.BlockSpec((1, BM, bd), lambda i, r: (i, r, 0)),
        out_shape=jax.ShapeDtypeStruct((nsup, n, bd), jnp.float32),
        compiler_params=pltpu.CompilerParams(
            dimension_semantics=("arbitrary", "arbitrary")),
    )(A, X0T)

    p = pl.pallas_call(
        functools.partial(_cheb2_body, bm=BM, nsup=float(nsup)),
        grid=(nb, nsup),
        in_specs=[
            pl.BlockSpec((1, BM, n), lambda r, i: (i, r, 0)),
            pl.BlockSpec((1, n, bd), lambda r, i: (i, 0, 0)),
            pl.BlockSpec((BM, bd), lambda r, i: (r, 0)),
        ],
        out_specs=pl.BlockSpec((BM, bd), lambda r, i: (r, 0)),
        out_shape=jax.ShapeDtypeStruct((n, bd), jnp.float32),
        compiler_params=pltpu.CompilerParams(
            dimension_semantics=("arbitrary", "arbitrary")),
    )(A, x1t, X0T)

    c = pl.pallas_call(
        functools.partial(_stats_body, bi=BI, bj=BJ, n=n),
        grid=(n // BJ,),
        in_specs=[pl.BlockSpec((n, Z.shape[1]), lambda j: (0, 0))],
        out_specs=pl.BlockSpec((1, BJ), lambda j: (0, j)),
        out_shape=jax.ShapeDtypeStruct((1, n), jnp.float32),
    )(Z)

    out = pl.pallas_call(
        functools.partial(_final_body, bi=BI, bj=BJ, n=n, batch=batch, d=d),
        grid=(n // BI,),
        in_specs=[
            pl.BlockSpec((n, Z.shape[1]), lambda r: (0, 0)),
            pl.BlockSpec((1, n), lambda r: (0, 0)),
            pl.BlockSpec((n, bd), lambda r: (0, 0)),
            pl.BlockSpec((BI, bd), lambda r: (r, 0)),
            pl.BlockSpec((d, out_f), lambda r: (0, 0)),
        ],
        out_specs=pl.BlockSpec((batch, BI, out_f), lambda r: (0, r, 0)),
        out_shape=jax.ShapeDtypeStruct((batch, n, out_f), jnp.float32),
    )(Z, c, X0T, p, W)

    return out


# speedup vs baseline: 1.1134x; 1.1134x over previous
import functools

import jax
import jax.numpy as jnp
from jax.experimental import pallas as pl
from jax.experimental.pallas import tpu as pltpu

_BF16 = jnp.bfloat16
_LOG2E = 1.4426950408889634


def _cheb1_body(a_ref, x0t_ref, out_ref):
    out_ref[0] = jnp.dot(a_ref[0].astype(_BF16), x0t_ref[...],
                         preferred_element_type=jnp.float32).astype(_BF16)


def _cheb2_body(a_ref, x1t_ref, x0t_ref, p_ref, *, bm, nsup):
    r = pl.program_id(0)
    i = pl.program_id(1)
    x1t = x1t_ref[0]
    rows = x1t_ref[0, pl.ds(r * bm, bm), :].astype(jnp.float32)
    acc = rows + 2.0 * jnp.dot(a_ref[0].astype(_BF16), x1t,
                               preferred_element_type=jnp.float32)

    @pl.when(i == 0)
    def _():
        p_ref[...] = acc + (1.0 - nsup) * x0t_ref[...]

    @pl.when(i != 0)
    def _():
        p_ref[...] += acc


def _stats_body(zib_ref, zjb_ref, negb_ref, ell_ref, *, bi, n):
    zjt = zjb_ref[...]
    nbj = negb_ref[...]

    def body(k, dacc):
        zit = zib_ref[pl.ds(k * bi, bi), :]
        t = jax.lax.dot_general(
            zit, zjt, (((1,), (1,)), ((), ())),
            preferred_element_type=jnp.float32)
        t2 = jnp.maximum(t, nbj)
        return dacc + jnp.sum(jnp.exp2(t2), axis=0, keepdims=True)

    d = jax.lax.fori_loop(0, n // bi, body,
                          jnp.zeros(nbj.shape, jnp.float32))
    ell_ref[...] = jnp.log2(d)


def _final_body(zib_ref, zjb_ref, ell_ref, negb_ref, x0t16_ref, p_ref,
                w_ref, out_ref, *, bi, bj, n, batch, d):
    zit = zib_ref[...]
    bd = batch * d

    def body(k, acc):
        zjt = zjb_ref[pl.ds(k * bj, bj), :]
        t = jax.lax.dot_general(
            zit, zjt, (((1,), (1,)), ((), ())),
            preferred_element_type=jnp.float32)
        lj = ell_ref[:, pl.ds(k * bj, bj)]
        fj = negb_ref[:, pl.ds(k * bj, bj)] - lj
        t2 = jnp.maximum(t - lj, fj)
        e = jnp.exp2(t2).astype(_BF16)
        v = x0t16_ref[pl.ds(k * bj, bj), :]
        return acc + jnp.dot(e, v, preferred_element_type=jnp.float32)

    xz = jax.lax.fori_loop(0, n // bj, body,
                           jnp.zeros((bi, bd), jnp.float32))
    s = xz + p_ref[...]
    w = w_ref[...]
    for b in range(batch):
        out_ref[b] = jnp.dot(s[:, b * d:(b + 1) * d], w,
                             preferred_element_type=jnp.float32)


def kernel(A, X, Z, W):
    nsup, n, _ = A.shape
    batch, d, _ = X.shape
    zdim = Z.shape[1]
    bd = batch * d
    out_f = W.shape[1]

    X0T = X.reshape(bd, n).T
    X0T16 = X0T.astype(_BF16)

    nrm2 = jnp.sum(Z * Z, axis=1)
    bbound = jnp.sqrt(nrm2 * jnp.max(nrm2))
    nb16 = (-bbound * _LOG2E).astype(_BF16)
    pad = jnp.zeros((n, 128 - zdim - 1), _BF16)
    zib = jnp.concatenate(
        [(Z * _LOG2E).astype(_BF16), jnp.ones((n, 1), _BF16), pad], axis=1)
    zjb = jnp.concatenate(
        [Z.astype(_BF16), nb16[:, None], pad], axis=1)
    negb = nb16.astype(jnp.float32)[None, :]
    zext = zib.shape[1]

    BM = 512
    BI = 512
    BJ = 512
    nb = n // BM

    x1t = pl.pallas_call(
        _cheb1_body,
        grid=(nsup, nb),
        in_specs=[
            pl.BlockSpec((1, BM, n), lambda i, r: (i, r, 0)),
            pl.BlockSpec((n, bd), lambda i, r: (0, 0)),
        ],
        out_specs=pl.BlockSpec((1, BM, bd), lambda i, r: (i, r, 0)),
        out_shape=jax.ShapeDtypeStruct((nsup, n, bd), _BF16),
        compiler_params=pltpu.CompilerParams(
            dimension_semantics=("arbitrary", "arbitrary")),
    )(A, X0T16)

    p = pl.pallas_call(
        functools.partial(_cheb2_body, bm=BM, nsup=float(nsup)),
        grid=(nb, nsup),
        in_specs=[
            pl.BlockSpec((1, BM, n), lambda r, i: (i, r, 0)),
            pl.BlockSpec((1, n, bd), lambda r, i: (i, 0, 0)),
            pl.BlockSpec((BM, bd), lambda r, i: (r, 0)),
        ],
        out_specs=pl.BlockSpec((BM, bd), lambda r, i: (r, 0)),
        out_shape=jax.ShapeDtypeStruct((n, bd), jnp.float32),
        compiler_params=pltpu.CompilerParams(
            dimension_semantics=("arbitrary", "arbitrary")),
    )(A, x1t, X0T)

    ell = pl.pallas_call(
        functools.partial(_stats_body, bi=BI, n=n),
        grid=(n // BJ,),
        in_specs=[
            pl.BlockSpec((n, zext), lambda j: (0, 0)),
            pl.BlockSpec((BJ, zext), lambda j: (j, 0)),
            pl.BlockSpec((1, BJ), lambda j: (0, j)),
        ],
        out_specs=pl.BlockSpec((1, BJ), lambda j: (0, j)),
        out_shape=jax.ShapeDtypeStruct((1, n), jnp.float32),
    )(zib, zjb, negb)

    out = pl.pallas_call(
        functools.partial(_final_body, bi=BI, bj=BJ, n=n, batch=batch, d=d),
        grid=(n // BI,),
        in_specs=[
            pl.BlockSpec((BI, zext), lambda r: (r, 0)),
            pl.BlockSpec((n, zext), lambda r: (0, 0)),
            pl.BlockSpec((1, n), lambda r: (0, 0)),
            pl.BlockSpec((1, n), lambda r: (0, 0)),
            pl.BlockSpec((n, bd), lambda r: (0, 0)),
            pl.BlockSpec((BI, bd), lambda r: (r, 0)),
            pl.BlockSpec((d, out_f), lambda r: (0, 0)),
        ],
        out_specs=pl.BlockSpec((batch, BI, out_f), lambda r: (0, r, 0)),
        out_shape=jax.ShapeDtypeStruct((batch, n, out_f), jnp.float32),
    )(zib, zjb, ell, negb, X0T16, p, W)

    return out


# speedup vs baseline: 1.1908x; 1.0695x over previous
import functools

import jax
import jax.numpy as jnp
from jax.experimental import pallas as pl
from jax.experimental.pallas import tpu as pltpu

_BF16 = jnp.bfloat16
_F8 = jnp.float8_e4m3fn
_LOG2E = 1.4426950408889634


def _cheb1_body(a_ref, x0t_ref, x1t_ref, a8_ref):
    a8 = a_ref[0].astype(_F8)
    a8_ref[0] = a8
    x1 = jnp.dot(a8, x0t_ref[...], preferred_element_type=jnp.float32)
    x1t_ref[0] = jnp.clip(x1, -440.0, 440.0).astype(_F8)


def _cheb2_body(a8_ref, x1t_ref, x0t_ref, p_ref, *, bm, nsup):
    r = pl.program_id(0)
    i = pl.program_id(1)
    x1t = x1t_ref[0]
    rows = x1t_ref[0, pl.ds(r * bm, bm), :].astype(jnp.float32)
    acc = rows + 2.0 * jnp.dot(a8_ref[0], x1t,
                               preferred_element_type=jnp.float32)

    @pl.when(i == 0)
    def _():
        p_ref[...] = acc + (1.0 - nsup) * x0t_ref[...]

    @pl.when(i != 0)
    def _():
        p_ref[...] += acc


def _stats_body(zib_ref, zjb_ref, negb_ref, ell_ref, *, bi, n):
    zjt = zjb_ref[...]
    nbj = negb_ref[...]

    def body(k, dacc):
        zit = zib_ref[pl.ds(k * bi, bi), :]
        t = jax.lax.dot_general(
            zit, zjt, (((1,), (1,)), ((), ())),
            preferred_element_type=jnp.float32)
        t2 = jnp.maximum(t, nbj)
        return dacc + jnp.sum(jnp.exp2(t2), axis=0, keepdims=True)

    d = jax.lax.fori_loop(0, n // bi, body,
                          jnp.zeros(nbj.shape, jnp.float32))
    ell_ref[...] = jnp.log2(d)


def _final_body(zib_ref, zjb_ref, ell_ref, negb_ref, x0t16_ref, p_ref,
                w_ref, out_ref, *, bi, bj, n, batch, d):
    zit = zib_ref[...]
    bd = batch * d

    def body(k, acc):
        zjt = zjb_ref[pl.ds(k * bj, bj), :]
        t = jax.lax.dot_general(
            zit, zjt, (((1,), (1,)), ((), ())),
            preferred_element_type=jnp.float32)
        lj = ell_ref[:, pl.ds(k * bj, bj)]
        fj = negb_ref[:, pl.ds(k * bj, bj)] - lj
        t2 = jnp.maximum(t - lj, fj)
        e = jnp.exp2(t2).astype(_BF16)
        v = x0t16_ref[pl.ds(k * bj, bj), :]
        return acc + jnp.dot(e, v, preferred_element_type=jnp.float32)

    xz = jax.lax.fori_loop(0, n // bj, body,
                           jnp.zeros((bi, bd), jnp.float32))
    s = xz + p_ref[...]
    w = w_ref[...]
    for b in range(batch):
        out_ref[b] = jnp.dot(s[:, b * d:(b + 1) * d], w,
                             preferred_element_type=jnp.float32)


def kernel(A, X, Z, W):
    nsup, n, _ = A.shape
    batch, d, _ = X.shape
    zdim = Z.shape[1]
    bd = batch * d
    out_f = W.shape[1]

    X0T = X.reshape(bd, n).T
    X0T16 = X0T.astype(_BF16)
    X0T8 = X0T.astype(_F8)

    nrm2 = jnp.sum(Z * Z, axis=1)
    bbound = jnp.sqrt(nrm2 * jnp.max(nrm2))
    nb16 = (-bbound * _LOG2E).astype(_BF16)
    pad = jnp.zeros((n, 128 - zdim - 1), _BF16)
    zib = jnp.concatenate(
        [(Z * _LOG2E).astype(_BF16), jnp.ones((n, 1), _BF16), pad], axis=1)
    zjb = jnp.concatenate(
        [Z.astype(_BF16), nb16[:, None], pad], axis=1)
    negb = nb16.astype(jnp.float32)[None, :]
    zext = zib.shape[1]

    BM = 512
    BI = 512
    BJ = 512
    nb = n // BM

    x1t, a8 = pl.pallas_call(
        _cheb1_body,
        grid=(nsup, nb),
        in_specs=[
            pl.BlockSpec((1, BM, n), lambda i, r: (i, r, 0)),
            pl.BlockSpec((n, bd), lambda i, r: (0, 0)),
        ],
        out_specs=[
            pl.BlockSpec((1, BM, bd), lambda i, r: (i, r, 0)),
            pl.BlockSpec((1, BM, n), lambda i, r: (i, r, 0)),
        ],
        out_shape=[
            jax.ShapeDtypeStruct((nsup, n, bd), _F8),
            jax.ShapeDtypeStruct((nsup, n, n), _F8),
        ],
        compiler_params=pltpu.CompilerParams(
            dimension_semantics=("arbitrary", "arbitrary")),
    )(A, X0T8)

    p = pl.pallas_call(
        functools.partial(_cheb2_body, bm=BM, nsup=float(nsup)),
        grid=(nb, nsup),
        in_specs=[
            pl.BlockSpec((1, BM, n), lambda r, i: (i, r, 0)),
            pl.BlockSpec((1, n, bd), lambda r, i: (i, 0, 0)),
            pl.BlockSpec((BM, bd), lambda r, i: (r, 0)),
        ],
        out_specs=pl.BlockSpec((BM, bd), lambda r, i: (r, 0)),
        out_shape=jax.ShapeDtypeStruct((n, bd), jnp.float32),
        compiler_params=pltpu.CompilerParams(
            dimension_semantics=("arbitrary", "arbitrary")),
    )(a8, x1t, X0T)

    ell = pl.pallas_call(
        functools.partial(_stats_body, bi=BI, n=n),
        grid=(n // BJ,),
        in_specs=[
            pl.BlockSpec((n, zext), lambda j: (0, 0)),
            pl.BlockSpec((BJ, zext), lambda j: (j, 0)),
            pl.BlockSpec((1, BJ), lambda j: (0, j)),
        ],
        out_specs=pl.BlockSpec((1, BJ), lambda j: (0, j)),
        out_shape=jax.ShapeDtypeStruct((1, n), jnp.float32),
    )(zib, zjb, negb)

    out = pl.pallas_call(
        functools.partial(_final_body, bi=BI, bj=BJ, n=n, batch=batch, d=d),
        grid=(n // BI,),
        in_specs=[
            pl.BlockSpec((BI, zext), lambda r: (r, 0)),
            pl.BlockSpec((n, zext), lambda r: (0, 0)),
            pl.BlockSpec((1, n), lambda r: (0, 0)),
            pl.BlockSpec((1, n), lambda r: (0, 0)),
            pl.BlockSpec((n, bd), lambda r: (0, 0)),
            pl.BlockSpec((BI, bd), lambda r: (r, 0)),
            pl.BlockSpec((d, out_f), lambda r: (0, 0)),
        ],
        out_specs=pl.BlockSpec((batch, BI, out_f), lambda r: (0, r, 0)),
        out_shape=jax.ShapeDtypeStruct((batch, n, out_f), jnp.float32),
    )(zib, zjb, ell, negb, X0T16, p, W)

    return out
